# baseline (speedup 1.0000x reference)
import functools
import jax
import jax.numpy as jnp
from jax import lax
from jax.experimental import pallas as pl
from jax.experimental.pallas import tpu as pltpu
from jax.experimental.pallas import tpu_sc as plsc


def _make_kernel(S, B, V, D, num_cores, num_subcores):
    NW = num_cores * num_subcores
    N = S * B
    VW = V // NW
    VC = 625
    n_vchunks = VW // VC
    WIN = 640
    C = 512
    CB = C // 128
    R = D // 8
    TP = C + 1
    per_w = (N // C) // NW
    bc_per_s = B // C

    mesh = plsc.VectorSubcoreMesh(core_axis_name="c", subcore_axis_name="s")

    @functools.partial(
        pl.kernel,
        mesh=mesh,
        out_type=(
            jax.ShapeDtypeStruct((S * R * (B // 128) * 8, 128), jnp.float32),
            jax.ShapeDtypeStruct((V, D), jnp.float32),
        ),
        scratch_types=[
            pltpu.SemaphoreType.REGULAR,
            [pltpu.SemaphoreType.DMA] * 2,
            [pltpu.SemaphoreType.DMA] * 2,
        ],
        compiler_params=pltpu.CompilerParams(
            use_tc_tiling_on_sc=False, needs_layout_passes=False
        ),
    )
    def k(idx_hbm, tabt_hbm, out_hbm, ts_hbm, xsem, asem, bsem):
        cid = lax.axis_index("c")
        wid = lax.axis_index("s") * num_cores + cid
        lanes = lax.iota(jnp.int32, 16)
        lanes_hi = lanes + 16
        lanes33 = lanes * 33

        def phase1(tin, tpad):
            v_base = wid * VW

            def read_win(c, b):
                v0 = v_base + c * VC
                v0a = (v0 // 16) * 16
                for d in range(D):
                    pltpu.async_copy(
                        tabt_hbm.at[pl.ds(d, 1), pl.ds(v0a, WIN)],
                        tin[b].at[pl.ds(d, 1), :],
                        asem[b],
                    )

            def wait_read(c, b):
                v0 = v_base + c * VC
                v0a = (v0 // 16) * 16
                for d in range(D):
                    pltpu.make_async_copy(
                        tabt_hbm.at[pl.ds(d, 1), pl.ds(v0a, WIN)],
                        tin[b].at[pl.ds(d, 1), :],
                        asem[b],
                    ).wait()

            def write_chunk(c, b):
                v0 = v_base + c * VC
                return pltpu.async_copy(
                    tpad[b].at[:, pl.ds(0, D)],
                    ts_hbm.at[pl.ds(v0, VC), :],
                    bsem[b],
                )

            def wait_write(c, b):
                v0 = v_base + c * VC
                pltpu.make_async_copy(
                    tpad[b].at[:, pl.ds(0, D)],
                    ts_hbm.at[pl.ds(v0, VC), :],
                    bsem[b],
                ).wait()

            def transpose_chunk(c, b):
                v0 = v_base + c * VC
                extra = v0 - (v0 // 16) * 16
                src = tin[b]
                dst = tpad[b]

                def g_body(jg, carry):
                    colv = extra + jg * 16 + lanes
                    rowv = lanes33 + jg * (16 * 33)
                    for d in range(D):
                        vec = plsc.load_gather(
                            src, (jnp.full_like(lanes, d), colv)
                        )
                        plsc.store_scatter(
                            dst, (jg * 16 + lanes, jnp.full_like(lanes, d)), vec
                        )
                    return carry

                lax.fori_loop(0, VC // 16, g_body, 0)
                tmask = lanes < (VC - (VC // 16) * 16)
                jg = VC // 16
                colv = extra + jg * 16 + lanes
                for d in range(D):
                    vec = plsc.load_gather(
                        src, (jnp.full_like(lanes, d), colv), mask=tmask
                    )
                    plsc.store_scatter(
                        dst,
                        (jg * 16 + lanes, jnp.full_like(lanes, d)),
                        vec,
                        mask=tmask,
                    )

            read_win(0, 0)
            read_win(1, 1)

            def body(i, carry):
                for b in range(2):
                    c = 2 * i + b
                    wait_read(c, b)

                    @pl.when(c >= 2)
                    def _():
                        wait_write(c - 2, b)

                    transpose_chunk(c, b)
                    write_chunk(c, b)

                    @pl.when(c + 2 < n_vchunks)
                    def _():
                        read_win(c + 2, b)

                return carry

            lax.fori_loop(0, n_vchunks // 2, body, 0)
            wait_write(n_vchunks - 2, 0)
            wait_write(n_vchunks - 1, 1)

        pl.run_scoped(
            phase1,
            [pltpu.VMEM((D, WIN), jnp.float32)] * 2,
            [pltpu.VMEM((VC, D + 1), jnp.float32)] * 2,
        )

        plsc.subcore_barrier()
        pl.semaphore_signal(xsem, 1, core_index=jnp.astype(1 - cid, jnp.int32))
        pl.semaphore_wait(xsem, 1)

        def phase2(idx_v, rows_v, tr_v):
            task0 = wid * per_w

            def load_idx(t, b):
                off = pl.multiple_of(t * C, 8)
                pltpu.sync_copy(idx_hbm.at[pl.ds(off, C)], idx_v[b])

            def gather(t, b):
                return pltpu.async_copy(
                    ts_hbm.at[idx_v[b]], rows_v[b], asem[b]
                )

            def wait_gather(t, b):
                pltpu.make_async_copy(
                    ts_hbm.at[idx_v[b]], rows_v[b], asem[b]
                ).wait()

            def write(t, b):
                s = t // bc_per_s
                bc = t % bc_per_s
                for r in range(R):
                    for c in range(CB):
                        rc = r * (B // 128) + bc * CB + c
                        dst = out_hbm.at[
                            pl.ds((s * (B // 128) * R + rc) * 8, 8), :
                        ]
                        src = tr_v[b].at[pl.ds(8 * r, 8), pl.ds(128 * c, 128)]
                        pltpu.async_copy(src, dst, bsem[b])

            def wait_write(t, b):
                s = t // bc_per_s
                bc = t % bc_per_s
                for r in range(R):
                    for c in range(CB):
                        rc = r * (B // 128) + bc * CB + c
                        dst = out_hbm.at[
                            pl.ds((s * (B // 128) * R + rc) * 8, 8), :
                        ]
                        src = tr_v[b].at[pl.ds(8 * r, 8), pl.ds(128 * c, 128)]
                        pltpu.make_async_copy(src, dst, bsem[b]).wait()

            def transpose(b):
                src = rows_v[b]
                dst = tr_v[b]

                def j_body(j0, carry):
                    for u in range(8):
                        j = j0 * 8 + u
                        col = jnp.full_like(lanes, j)
                        v_lo = plsc.load_gather(src, (col, lanes))
                        v_hi = plsc.load_gather(src, (col, lanes_hi))
                        plsc.store_scatter(dst, (lanes, col), v_lo)
                        plsc.store_scatter(dst, (lanes_hi, col), v_hi)
                    return carry

                lax.fori_loop(0, C // 8, j_body, 0)

            load_idx(task0, 0)
            gather(task0, 0)
            load_idx(task0 + 1, 1)
            gather(task0 + 1, 1)

            def body(i, carry):
                for b in range(2):
                    t = task0 + 2 * i + b
                    wait_gather(t, b)

                    @pl.when(2 * i + b >= 2)
                    def _():
                        wait_write(t - 2, b)

                    transpose(b)
                    write(t, b)

                    @pl.when(2 * i + b + 2 < per_w)
                    def _():
                        load_idx(t + 2, b)
                        gather(t + 2, b)

                return carry

            lax.fori_loop(0, per_w // 2, body, 0)
            wait_write(task0 + per_w - 2, 0)
            wait_write(task0 + per_w - 1, 1)

        pl.run_scoped(
            phase2,
            [pltpu.VMEM((C,), jnp.int32)] * 2,
            [pltpu.VMEM((C, D), jnp.float32)] * 2,
            [pltpu.VMEM((D, TP), jnp.float32)] * 2,
        )

    return k


def kernel(x, table):
    B, S = x.shape
    V, D = table.shape
    info = plsc.get_sparse_core_info()
    k = _make_kernel(S, B, V, D, info.num_cores, info.num_subcores)
    xt = x.T.reshape(S * B).astype(jnp.int32)
    tabt = table.T
    out, _ = k(xt, tabt)
    out5 = out.reshape(S, D // 8, B // 128, 8, 128)
    return out5.transpose(2, 4, 0, 1, 3).reshape(B, S, D)


# speedup vs baseline: 1.0452x; 1.0452x over previous
import functools
import jax
import jax.numpy as jnp
from jax import lax
from jax.experimental import pallas as pl
from jax.experimental.pallas import tpu as pltpu
from jax.experimental.pallas import tpu_sc as plsc


def _make_kernel(S, B, V, D, num_cores, num_subcores):
    NW = num_cores * num_subcores
    N = S * B
    VW = V // NW
    VC = 625
    n_vchunks = VW // VC
    WIN = 640
    C = 512
    CB = C // 128
    R = D // 8
    TP = C + 1
    per_w = (N // C) // NW
    bc_per_s = B // C

    mesh = plsc.VectorSubcoreMesh(core_axis_name="c", subcore_axis_name="s")

    @functools.partial(
        pl.kernel,
        mesh=mesh,
        out_type=(
            jax.ShapeDtypeStruct((S * R * (B // 128) * 8, 128), jnp.float32),
            jax.ShapeDtypeStruct((V, D + 1), jnp.float32),
        ),
        scratch_types=[
            pltpu.SemaphoreType.REGULAR,
            [pltpu.SemaphoreType.DMA] * 2,
            [pltpu.SemaphoreType.DMA] * 2,
        ],
        compiler_params=pltpu.CompilerParams(
            use_tc_tiling_on_sc=False, needs_layout_passes=False
        ),
    )
    def k(idx_hbm, tabt_hbm, out_hbm, ts_hbm, xsem, asem, bsem):
        cid = lax.axis_index("c")
        wid = lax.axis_index("s") * num_cores + cid
        lanes = lax.iota(jnp.int32, 16)
        lanes_hi = lanes + 16
        lanes33 = lanes * 33

        def phase1(tin, tpad):
            v_base = wid * VW

            def read_win(c, b):
                v0 = v_base + c * VC
                v0a = (v0 // 16) * 16
                for d in range(D):
                    pltpu.async_copy(
                        tabt_hbm.at[pl.ds(d, 1), pl.ds(v0a, WIN)],
                        tin[b].at[pl.ds(d, 1), :],
                        asem[b],
                    )

            def wait_read(c, b):
                v0 = v_base + c * VC
                v0a = (v0 // 16) * 16
                for d in range(D):
                    pltpu.make_async_copy(
                        tabt_hbm.at[pl.ds(d, 1), pl.ds(v0a, WIN)],
                        tin[b].at[pl.ds(d, 1), :],
                        asem[b],
                    ).wait()

            def write_chunk(c, b):
                v0 = v_base + c * VC
                return pltpu.async_copy(
                    tpad[b], ts_hbm.at[pl.ds(v0, VC), :], bsem[b]
                )

            def wait_write(c, b):
                v0 = v_base + c * VC
                pltpu.make_async_copy(
                    tpad[b], ts_hbm.at[pl.ds(v0, VC), :], bsem[b]
                ).wait()

            def transpose_chunk(c, b):
                v0 = v_base + c * VC
                extra = v0 - (v0 // 16) * 16
                src = tin[b]
                dst = tpad[b]

                def g_body(jg, carry):
                    colv = extra + jg * 16 + lanes
                    rowv = lanes33 + jg * (16 * 33)
                    for d in range(D):
                        vec = plsc.load_gather(
                            src, (jnp.full_like(lanes, d), colv)
                        )
                        plsc.store_scatter(
                            dst, (jg * 16 + lanes, jnp.full_like(lanes, d)), vec
                        )
                    return carry

                lax.fori_loop(0, VC // 16, g_body, 0)
                tmask = lanes < (VC - (VC // 16) * 16)
                jg = VC // 16
                colv = extra + jg * 16 + lanes
                for d in range(D):
                    vec = plsc.load_gather(
                        src, (jnp.full_like(lanes, d), colv), mask=tmask
                    )
                    plsc.store_scatter(
                        dst,
                        (jg * 16 + lanes, jnp.full_like(lanes, d)),
                        vec,
                        mask=tmask,
                    )

            read_win(0, 0)
            read_win(1, 1)

            def body(i, carry):
                for b in range(2):
                    c = 2 * i + b
                    wait_read(c, b)

                    @pl.when(c >= 2)
                    def _():
                        wait_write(c - 2, b)

                    transpose_chunk(c, b)
                    write_chunk(c, b)

                    @pl.when(c + 2 < n_vchunks)
                    def _():
                        read_win(c + 2, b)

                return carry

            lax.fori_loop(0, n_vchunks // 2, body, 0)
            wait_write(n_vchunks - 2, 0)
            wait_write(n_vchunks - 1, 1)

        pl.run_scoped(
            phase1,
            [pltpu.VMEM((D, WIN), jnp.float32)] * 2,
            [pltpu.VMEM((VC, D + 1), jnp.float32)] * 2,
        )

        plsc.subcore_barrier()
        pl.semaphore_signal(xsem, 1, core_index=jnp.astype(1 - cid, jnp.int32))
        pl.semaphore_wait(xsem, 1)

        def phase2(idx_v, rows_v, tr_v):
            task0 = wid * per_w

            def load_idx(t, b):
                off = pl.multiple_of(t * C, 8)
                pltpu.sync_copy(idx_hbm.at[pl.ds(off, C)], idx_v[b])

            def gather(t, b):
                return pltpu.async_copy(
                    ts_hbm.at[idx_v[b]], rows_v[b], asem[b]
                )

            def wait_gather(t, b):
                pltpu.make_async_copy(
                    ts_hbm.at[idx_v[b]], rows_v[b], asem[b]
                ).wait()

            def write(t, b):
                s = t // bc_per_s
                bc = t % bc_per_s
                for r in range(R):
                    for c in range(CB):
                        rc = r * (B // 128) + bc * CB + c
                        dst = out_hbm.at[
                            pl.ds((s * (B // 128) * R + rc) * 8, 8), :
                        ]
                        src = tr_v[b].at[pl.ds(8 * r, 8), pl.ds(128 * c, 128)]
                        pltpu.async_copy(src, dst, bsem[b])

            def wait_write(t, b):
                s = t // bc_per_s
                bc = t % bc_per_s
                for r in range(R):
                    for c in range(CB):
                        rc = r * (B // 128) + bc * CB + c
                        dst = out_hbm.at[
                            pl.ds((s * (B // 128) * R + rc) * 8, 8), :
                        ]
                        src = tr_v[b].at[pl.ds(8 * r, 8), pl.ds(128 * c, 128)]
                        pltpu.make_async_copy(src, dst, bsem[b]).wait()

            def transpose(b):
                src = rows_v[b]
                dst = tr_v[b]

                def j_body(j0, carry):
                    for u in range(8):
                        j = j0 * 8 + u
                        col = jnp.full_like(lanes, j)
                        v_lo = plsc.load_gather(src, (col, lanes))
                        v_hi = plsc.load_gather(src, (col, lanes_hi))
                        plsc.store_scatter(dst, (lanes, col), v_lo)
                        plsc.store_scatter(dst, (lanes_hi, col), v_hi)
                    return carry

                lax.fori_loop(0, C // 8, j_body, 0)

            load_idx(task0, 0)
            gather(task0, 0)
            load_idx(task0 + 1, 1)
            gather(task0 + 1, 1)

            def body(i, carry):
                for b in range(2):
                    t = task0 + 2 * i + b
                    wait_gather(t, b)

                    @pl.when(2 * i + b >= 2)
                    def _():
                        wait_write(t - 2, b)

                    transpose(b)
                    write(t, b)

                    @pl.when(2 * i + b + 2 < per_w)
                    def _():
                        load_idx(t + 2, b)
                        gather(t + 2, b)

                return carry

            lax.fori_loop(0, per_w // 2, body, 0)
            wait_write(task0 + per_w - 2, 0)
            wait_write(task0 + per_w - 1, 1)

        pl.run_scoped(
            phase2,
            [pltpu.VMEM((C,), jnp.int32)] * 2,
            [pltpu.VMEM((C, D + 1), jnp.float32)] * 2,
            [pltpu.VMEM((D, TP), jnp.float32)] * 2,
        )

    return k


def kernel(x, table):
    B, S = x.shape
    V, D = table.shape
    info = plsc.get_sparse_core_info()
    k = _make_kernel(S, B, V, D, info.num_cores, info.num_subcores)
    xt = x.T.reshape(S * B).astype(jnp.int32)
    tabt = table.T
    out, _ = k(xt, tabt)
    out5 = out.reshape(S, D // 8, B // 128, 8, 128)
    return out5.transpose(2, 4, 0, 1, 3).reshape(B, S, D)


# speedup vs baseline: 4.4210x; 4.2297x over previous
import functools
import jax
import jax.numpy as jnp
from jax import lax
from jax.experimental import pallas as pl
from jax.experimental.pallas import tpu as pltpu
from jax.experimental.pallas import tpu_sc as plsc


def _make_gather(S, B, V, D, num_cores, num_subcores):
    NW = num_cores * num_subcores
    N = S * B
    C = 512
    CB = C // 128
    R = D // 8
    TP = C + 1
    n_tasks = N // C
    per_w = n_tasks // NW
    bc_per_s = B // C

    mesh = plsc.VectorSubcoreMesh(core_axis_name="c", subcore_axis_name="s")

    @functools.partial(
        pl.kernel,
        mesh=mesh,
        out_type=jax.ShapeDtypeStruct((S * R * (B // 128) * 8, 128), jnp.float32),
        scratch_types=[
            pltpu.VMEM((per_w * C,), jnp.int32),
            [pltpu.VMEM((C, D), jnp.float32)] * 2,
            [pltpu.VMEM((D, TP), jnp.float32)] * 2,
            [pltpu.SemaphoreType.DMA] * 2,
            [pltpu.SemaphoreType.DMA] * 2,
        ],
        compiler_params=pltpu.CompilerParams(
            use_tc_tiling_on_sc=False, needs_layout_passes=False
        ),
    )
    def k(idx_hbm, table_hbm, out_hbm, idx_v, rows_v, tr_v, gsem, wsem):
        wid = lax.axis_index("s") * num_cores + lax.axis_index("c")
        task0 = wid * per_w

        lanes = lax.iota(jnp.int32, 16)
        lanes_hi = lanes + 16

        pltpu.sync_copy(
            idx_hbm.at[pl.ds(pl.multiple_of(task0 * C, 8), per_w * C)], idx_v
        )

        def gather(t, b):
            off = pl.multiple_of((t - task0) * C, 8)
            return pltpu.async_copy(
                table_hbm.at[idx_v.at[pl.ds(off, C)]], rows_v[b], gsem[b]
            )

        def wait_gather(t, b):
            off = pl.multiple_of((t - task0) * C, 8)
            pltpu.make_async_copy(
                table_hbm.at[idx_v.at[pl.ds(off, C)]], rows_v[b], gsem[b]
            ).wait()

        def write(t, b):
            s = t // bc_per_s
            bc = t % bc_per_s
            for r in range(R):
                for c in range(CB):
                    rc = r * (B // 128) + bc * CB + c
                    dst = out_hbm.at[pl.ds((s * (B // 128) * R + rc) * 8, 8), :]
                    src = tr_v[b].at[pl.ds(8 * r, 8), pl.ds(128 * c, 128)]
                    pltpu.async_copy(src, dst, wsem[b])

        def wait_write(t, b):
            s = t // bc_per_s
            bc = t % bc_per_s
            for r in range(R):
                for c in range(CB):
                    rc = r * (B // 128) + bc * CB + c
                    dst = out_hbm.at[pl.ds((s * (B // 128) * R + rc) * 8, 8), :]
                    src = tr_v[b].at[pl.ds(8 * r, 8), pl.ds(128 * c, 128)]
                    pltpu.make_async_copy(src, dst, wsem[b]).wait()

        def transpose(b):
            src = rows_v[b]
            dst = tr_v[b]

            def j_body(j0, carry):
                for u in range(8):
                    j = j0 * 8 + u
                    col = jnp.full_like(lanes, j)
                    v_lo = plsc.load_gather(src, (col, lanes))
                    v_hi = plsc.load_gather(src, (col, lanes_hi))
                    plsc.store_scatter(dst, (lanes, col), v_lo)
                    plsc.store_scatter(dst, (lanes_hi, col), v_hi)
                return carry

            lax.fori_loop(0, C // 8, j_body, 0)

        gather(task0, 0)
        gather(task0 + 1, 1)

        def body(i, carry):
            for b in range(2):
                t = task0 + 2 * i + b
                wait_gather(t, b)

                @pl.when(2 * i + b >= 2)
                def _():
                    wait_write(t - 2, b)

                transpose(b)
                write(t, b)

                @pl.when(2 * i + b + 2 < per_w)
                def _():
                    gather(t + 2, b)

            return carry

        lax.fori_loop(0, per_w // 2, body, 0)

        wait_write(task0 + per_w - 2, 0)
        wait_write(task0 + per_w - 1, 1)

    return k


def kernel(x, table):
    B, S = x.shape
    V, D = table.shape
    info = plsc.get_sparse_core_info()
    k = _make_gather(S, B, V, D, info.num_cores, info.num_subcores)
    xt = x.T.reshape(S * B).astype(jnp.int32)
    out = k(xt, table)
    out5 = out.reshape(S, D // 8, B // 128, 8, 128)
    return out5.transpose(2, 4, 0, 1, 3).reshape(B, S, D)


# speedup vs baseline: 4.4286x; 1.0017x over previous
import functools
import jax
import jax.numpy as jnp
from jax import lax
from jax.experimental import pallas as pl
from jax.experimental.pallas import tpu as pltpu
from jax.experimental.pallas import tpu_sc as plsc


def _make_gather(S, B, V, D, num_cores, num_subcores):
    NW = num_cores * num_subcores
    N = S * B
    C = 512
    CB = C // 128
    R = D // 8
    TP = C + 1
    n_tasks = N // C
    per_w = n_tasks // NW
    bc_per_s = B // C

    mesh = plsc.VectorSubcoreMesh(core_axis_name="c", subcore_axis_name="s")

    @functools.partial(
        pl.kernel,
        mesh=mesh,
        out_type=jax.ShapeDtypeStruct((S * R * (B // 128) * 8, 128), jnp.float32),
        scratch_types=[
            pltpu.VMEM((per_w * C,), jnp.int32),
            [pltpu.VMEM((C, D), jnp.float32)] * 2,
            [pltpu.VMEM((D, TP), jnp.float32)] * 2,
            [pltpu.SemaphoreType.DMA] * 2,
            [pltpu.SemaphoreType.DMA] * 2,
        ],
        compiler_params=pltpu.CompilerParams(
            use_tc_tiling_on_sc=False, needs_layout_passes=False
        ),
    )
    def k(idx_hbm, table_hbm, out_hbm, idx_v, rows_v, tr_v, gsem, wsem):
        wid = lax.axis_index("s") * num_cores + lax.axis_index("c")
        task0 = wid * per_w

        lanes = lax.iota(jnp.int32, 16)
        lanes_hi = lanes + 16

        pltpu.sync_copy(
            idx_hbm.at[pl.ds(pl.multiple_of(task0 * C, 8), per_w * C)], idx_v
        )

        def gather(t, b):
            off = pl.multiple_of((t - task0) * C, 8)
            return pltpu.async_copy(
                table_hbm.at[idx_v.at[pl.ds(off, C)]], rows_v[b], gsem[b]
            )

        def wait_gather(t, b):
            off = pl.multiple_of((t - task0) * C, 8)
            pltpu.make_async_copy(
                table_hbm.at[idx_v.at[pl.ds(off, C)]], rows_v[b], gsem[b]
            ).wait()

        def write(t, b):
            s = t // bc_per_s
            bc = t % bc_per_s
            for r in range(R):
                for c in range(CB):
                    rc = r * (B // 128) + bc * CB + c
                    dst = out_hbm.at[pl.ds((s * (B // 128) * R + rc) * 8, 8), :]
                    src = tr_v[b].at[pl.ds(8 * r, 8), pl.ds(128 * c, 128)]
                    pltpu.async_copy(src, dst, wsem[b])

        def wait_write(t, b):
            s = t // bc_per_s
            bc = t % bc_per_s
            for r in range(R):
                for c in range(CB):
                    rc = r * (B // 128) + bc * CB + c
                    dst = out_hbm.at[pl.ds((s * (B // 128) * R + rc) * 8, 8), :]
                    src = tr_v[b].at[pl.ds(8 * r, 8), pl.ds(128 * c, 128)]
                    pltpu.make_async_copy(src, dst, wsem[b]).wait()

        def transpose(b):
            src = rows_v[b]
            dst = tr_v[b]

            def j_body(j0, carry):
                for u in range(16):
                    j = j0 * 16 + u
                    col = jnp.full_like(lanes, j)
                    v_lo = plsc.load_gather(src, (col, lanes))
                    v_hi = plsc.load_gather(src, (col, lanes_hi))
                    plsc.store_scatter(dst, (lanes, col), v_lo)
                    plsc.store_scatter(dst, (lanes_hi, col), v_hi)
                return carry

            lax.fori_loop(0, C // 16, j_body, 0)

        gather(task0, 0)
        gather(task0 + 1, 1)

        def body(i, carry):
            for b in range(2):
                t = task0 + 2 * i + b
                wait_gather(t, b)

                @pl.when(2 * i + b >= 2)
                def _():
                    wait_write(t - 2, b)

                transpose(b)
                write(t, b)

                @pl.when(2 * i + b + 2 < per_w)
                def _():
                    gather(t + 2, b)

            return carry

        lax.fori_loop(0, per_w // 2, body, 0)

        wait_write(task0 + per_w - 2, 0)
        wait_write(task0 + per_w - 1, 1)

    return k


def kernel(x, table):
    B, S = x.shape
    V, D = table.shape
    info = plsc.get_sparse_core_info()
    k = _make_gather(S, B, V, D, info.num_cores, info.num_subcores)
    xt = x.T.reshape(S * B).astype(jnp.int32)
    out = k(xt, table)
    out5 = out.reshape(S, D // 8, B // 128, 8, 128)
    return out5.transpose(2, 4, 0, 1, 3).reshape(B, S, D)
